# baseline (speedup 1.0000x reference)
import functools

import jax
import jax.numpy as jnp
from jax import lax
from jax.experimental import pallas as pl
from jax.experimental.pallas import tpu as pltpu
from jax.experimental.pallas import tpu_sc as plsc

N_NODES = 10000
N_EDGES = 320000
D = 128
DH = D // 2

NC = 2
NS = 16
B = 128
E_SELF = N_EDGES + N_NODES
K = 162
E_PAD = NS * K * B
N_ACC = 10240
ROWS_PER_TILE_OUT = N_ACC // NS
ZROWS = 128

_mesh = plsc.VectorSubcoreMesh(core_axis_name="c", subcore_axis_name="s")


@functools.partial(
    pl.kernel,
    out_type=(
        jax.ShapeDtypeStruct((N_ACC, DH), jnp.float32),
        jax.ShapeDtypeStruct((N_ACC, DH), jnp.float32),
    ),
    mesh=_mesh,
    scratch_types=[
        pltpu.VMEM((K, B), jnp.int32),
        pltpu.VMEM((K, B), jnp.int32),
        pltpu.VMEM((4, B, DH), jnp.float32),
        pltpu.VMEM((ZROWS, DH), jnp.float32),
        pltpu.VMEM_SHARED((N_ACC, DH), jnp.float32),
        pltpu.SemaphoreType.DMA,
        pltpu.SemaphoreType.DMA,
    ],
    compiler_params=pltpu.CompilerParams(use_tc_tiling_on_sc=False),
)
def _sc_aggregate(xl_hbm, xr_hbm, src_hbm, dst_hbm, pa_hbm, pb_hbm,
                  src_v, dst_v, rows_v, zbuf, acc_sh, sem_g, sem_s):
    c = lax.axis_index("c")
    s = lax.axis_index("s")

    pltpu.async_copy(src_hbm.at[s], src_v, sem_g)
    pltpu.async_copy(dst_hbm.at[s], dst_v, sem_g)

    zeros16 = jnp.zeros((16,), jnp.float32)

    def _zrow(i, carry):
        for c8 in range(DH // 16):
            zbuf[i, pl.ds(c8 * 16, 16)] = zeros16
        return carry

    lax.fori_loop(0, ZROWS, _zrow, 0)
    acc_rows_per_tile = N_ACC // NS
    nz = acc_rows_per_tile // ZROWS
    for t in range(nz):
        pltpu.async_copy(
            zbuf, acc_sh.at[pl.ds(s * acc_rows_per_tile + t * ZROWS, ZROWS)],
            sem_s)
    for ref in (src_v, dst_v):
        pltpu.make_async_copy(src_hbm.at[s], ref, sem_g).wait()
    for t in range(nz):
        pltpu.make_async_copy(
            zbuf, acc_sh.at[pl.ds(0, ZROWS)], sem_s).wait()
    plsc.subcore_barrier()

    NBUF = 4

    def _gather(j, buf):
        @pl.when(c == 0)
        def _():
            pltpu.async_copy(xl_hbm.at[src_v.at[j]], rows_v.at[buf], sem_g)

        @pl.when(c == 1)
        def _():
            pltpu.async_copy(xr_hbm.at[src_v.at[j]], rows_v.at[buf], sem_g)

    for j in range(NBUF - 1):
        _gather(j, j)

    def _block(j, carry):
        b = lax.rem(j, NBUF)
        pltpu.make_async_copy(
            xl_hbm.at[src_v.at[j]], rows_v.at[b], sem_g).wait()
        pltpu.async_copy(rows_v.at[b], acc_sh.at[dst_v.at[j]], sem_s,
                         add=True)

        @pl.when(j + NBUF - 1 < K)
        def _():
            @pl.when(j >= 1)
            def _():
                pltpu.make_async_copy(
                    rows_v.at[b], acc_sh.at[pl.ds(0, B)], sem_s).wait()

            _gather(j + NBUF - 1, lax.rem(j + NBUF - 1, NBUF))

        return carry

    lax.fori_loop(0, K, _block, 0)
    for _ in range(NBUF):
        pltpu.make_async_copy(
            rows_v.at[0], acc_sh.at[pl.ds(0, B)], sem_s).wait()
    plsc.subcore_barrier()

    base = s * ROWS_PER_TILE_OUT

    @pl.when(c == 0)
    def _():
        pltpu.sync_copy(acc_sh.at[pl.ds(base, ROWS_PER_TILE_OUT)],
                        pa_hbm.at[pl.ds(base, ROWS_PER_TILE_OUT)])

    @pl.when(c == 1)
    def _():
        pltpu.sync_copy(acc_sh.at[pl.ds(base, ROWS_PER_TILE_OUT)],
                        pb_hbm.at[pl.ds(base, ROWS_PER_TILE_OUT)])


def _mm_body(a_ref, b_ref, w_ref, bias_ref, ol_ref, or_ref, *, relu):
    a2 = a_ref[...]
    b2 = b_ref[...]
    aggr_e = jnp.concatenate([a2[:, :DH], b2[:, :DH]], axis=1)
    aggr_o = jnp.concatenate([a2[:, DH:], b2[:, DH:]], axis=1)
    w = w_ref[...]
    bias = bias_ref[...]

    def lin(aggr):
        y = jnp.dot(aggr, w, preferred_element_type=jnp.float32) + bias
        return jnp.maximum(y, 0.0) if relu else y

    ye = lin(aggr_e)
    yo = lin(aggr_o)
    ol_ref[...] = jnp.concatenate([ye[:, :DH], yo[:, :DH]], axis=1)
    or_ref[...] = jnp.concatenate([ye[:, DH:], yo[:, DH:]], axis=1)


_NP = N_NODES // 2


def _tc_linear(pa2, pb2, W, bias, relu):
    rows = 1000
    blk = pl.BlockSpec((rows, D), lambda i: (i, 0))
    return pl.pallas_call(
        functools.partial(_mm_body, relu=relu),
        grid=(_NP // rows,),
        in_specs=[blk, blk,
                  pl.BlockSpec((D, D), lambda i: (0, 0)),
                  pl.BlockSpec((1, D), lambda i: (0, 0))],
        out_specs=(blk, blk),
        out_shape=tuple(
            jax.ShapeDtypeStruct((_NP, D), jnp.float32) for _ in range(2)),
    )(pa2, pb2, W, bias)


def kernel(x, edge_index, W1, b1, W2, b2, W3, b3):
    ei = edge_index.astype(jnp.int32)
    pad = E_PAD - E_SELF
    loop = jnp.arange(N_NODES, dtype=jnp.int32)
    src_p = jnp.concatenate(
        [ei[0], loop, jnp.zeros((pad,), jnp.int32)]).reshape(NS, K, B)
    dst_p = jnp.concatenate(
        [ei[1], loop, jnp.full((pad,), N_NODES, jnp.int32)]).reshape(NS, K, B)

    hl = x[:, :DH]
    hr = x[:, DH:]
    for W, bias, relu in ((W1, b1, True), (W2, b2, True), (W3, b3, False)):
        pa, pb = _sc_aggregate(hl, hr, src_p, dst_p)
        ol, orr = _tc_linear(
            pa.reshape(N_ACC // 2, D), pb.reshape(N_ACC // 2, D),
            W, bias.reshape(1, D), relu)
        hl = ol.reshape(N_NODES, DH)
        hr = orr.reshape(N_NODES, DH)
    return jnp.concatenate([hl, hr], axis=1)


# speedup vs baseline: 1.0335x; 1.0335x over previous
import functools

import jax
import jax.numpy as jnp
from jax import lax
from jax.experimental import pallas as pl
from jax.experimental.pallas import tpu as pltpu
from jax.experimental.pallas import tpu_sc as plsc

N_NODES = 10000
N_EDGES = 320000
D = 128
DH = D // 2

NC = 2
NS = 16
B = 128
K = 157
E_PAD = NS * K * B
N_ACC = 10240
ROWS_PER_TILE_OUT = N_ACC // NS
ZROWS = 128

_mesh = plsc.VectorSubcoreMesh(core_axis_name="c", subcore_axis_name="s")


@functools.partial(
    pl.kernel,
    out_type=(
        jax.ShapeDtypeStruct((N_ACC, DH), jnp.float32),
        jax.ShapeDtypeStruct((N_ACC, DH), jnp.float32),
    ),
    mesh=_mesh,
    scratch_types=[
        pltpu.VMEM((K, B), jnp.int32),
        pltpu.VMEM((K, B), jnp.int32),
        pltpu.VMEM((4, B, DH), jnp.float32),
        pltpu.VMEM((ZROWS, DH), jnp.float32),
        pltpu.VMEM_SHARED((N_ACC, DH), jnp.float32),
        pltpu.SemaphoreType.DMA,
        pltpu.SemaphoreType.DMA,
    ],
    compiler_params=pltpu.CompilerParams(use_tc_tiling_on_sc=False),
)
def _sc_aggregate(xl_hbm, xr_hbm, src_hbm, dst_hbm, pa_hbm, pb_hbm,
                  src_v, dst_v, rows_v, zbuf, acc_sh, sem_g, sem_s):
    c = lax.axis_index("c")
    s = lax.axis_index("s")

    pltpu.async_copy(src_hbm.at[s], src_v, sem_g)
    pltpu.async_copy(dst_hbm.at[s], dst_v, sem_g)

    zeros16 = jnp.zeros((16,), jnp.float32)

    def _zrow(i, carry):
        for c8 in range(DH // 16):
            zbuf[i, pl.ds(c8 * 16, 16)] = zeros16
        return carry

    lax.fori_loop(0, ZROWS, _zrow, 0)
    acc_rows_per_tile = N_ACC // NS
    nz = acc_rows_per_tile // ZROWS
    for t in range(nz):
        pltpu.async_copy(
            zbuf, acc_sh.at[pl.ds(s * acc_rows_per_tile + t * ZROWS, ZROWS)],
            sem_s)
    for ref in (src_v, dst_v):
        pltpu.make_async_copy(src_hbm.at[s], ref, sem_g).wait()
    for t in range(nz):
        pltpu.make_async_copy(
            zbuf, acc_sh.at[pl.ds(0, ZROWS)], sem_s).wait()
    plsc.subcore_barrier()

    NBUF = 4

    def _gather(j, buf):
        @pl.when(c == 0)
        def _():
            pltpu.async_copy(xl_hbm.at[src_v.at[j]], rows_v.at[buf], sem_g)

        @pl.when(c == 1)
        def _():
            pltpu.async_copy(xr_hbm.at[src_v.at[j]], rows_v.at[buf], sem_g)

    for j in range(NBUF - 1):
        _gather(j, j)

    def _block(j, carry):
        b = lax.rem(j, NBUF)
        pltpu.make_async_copy(
            xl_hbm.at[src_v.at[j]], rows_v.at[b], sem_g).wait()
        pltpu.async_copy(rows_v.at[b], acc_sh.at[dst_v.at[j]], sem_s,
                         add=True)

        @pl.when(j + NBUF - 1 < K)
        def _():
            @pl.when(j >= 1)
            def _():
                pltpu.make_async_copy(
                    rows_v.at[b], acc_sh.at[pl.ds(0, B)], sem_s).wait()

            _gather(j + NBUF - 1, lax.rem(j + NBUF - 1, NBUF))

        return carry

    lax.fori_loop(0, K, _block, 0)
    for _ in range(NBUF):
        pltpu.make_async_copy(
            rows_v.at[0], acc_sh.at[pl.ds(0, B)], sem_s).wait()
    plsc.subcore_barrier()

    base = s * ROWS_PER_TILE_OUT

    @pl.when(c == 0)
    def _():
        pltpu.sync_copy(acc_sh.at[pl.ds(base, ROWS_PER_TILE_OUT)],
                        pa_hbm.at[pl.ds(base, ROWS_PER_TILE_OUT)])

    @pl.when(c == 1)
    def _():
        pltpu.sync_copy(acc_sh.at[pl.ds(base, ROWS_PER_TILE_OUT)],
                        pb_hbm.at[pl.ds(base, ROWS_PER_TILE_OUT)])


def _mm_body(xe_ref, xo_ref, a_ref, b_ref, w_ref, bias_ref,
             oe_ref, oo_ref, ol_ref, or_ref, *, relu):
    a2 = a_ref[...]
    b2 = b_ref[...]
    aggr_e = jnp.concatenate([a2[:, :DH], b2[:, :DH]], axis=1)
    aggr_o = jnp.concatenate([a2[:, DH:], b2[:, DH:]], axis=1)
    w = w_ref[...]
    bias = bias_ref[...]

    def lin(x, aggr):
        y = jnp.dot(x + aggr, w, preferred_element_type=jnp.float32) + bias
        return jnp.maximum(y, 0.0) if relu else y

    ye = lin(xe_ref[...], aggr_e)
    yo = lin(xo_ref[...], aggr_o)
    oe_ref[...] = ye
    oo_ref[...] = yo
    ol_ref[...] = jnp.concatenate([ye[:, :DH], yo[:, :DH]], axis=1)
    or_ref[...] = jnp.concatenate([ye[:, DH:], yo[:, DH:]], axis=1)


_NP = N_NODES // 2


def _tc_linear(xe, xo, pa2, pb2, W, bias, relu):
    rows = 1000
    blk = pl.BlockSpec((rows, D), lambda i: (i, 0))
    return pl.pallas_call(
        functools.partial(_mm_body, relu=relu),
        grid=(_NP // rows,),
        in_specs=[blk, blk, blk, blk,
                  pl.BlockSpec((D, D), lambda i: (0, 0)),
                  pl.BlockSpec((1, D), lambda i: (0, 0))],
        out_specs=(blk, blk, blk, blk),
        out_shape=tuple(
            jax.ShapeDtypeStruct((_NP, D), jnp.float32) for _ in range(4)),
    )(xe, xo, pa2, pb2, W, bias)


def kernel(x, edge_index, W1, b1, W2, b2, W3, b3):
    ei = edge_index.astype(jnp.int32)
    pad = E_PAD - N_EDGES
    src_p = jnp.concatenate(
        [ei[0], jnp.zeros((pad,), jnp.int32)]).reshape(NS, K, B)
    dst_p = jnp.concatenate(
        [ei[1], jnp.full((pad,), N_NODES, jnp.int32)]).reshape(NS, K, B)

    xe = x[0::2]
    xo = x[1::2]
    hl = x[:, :DH]
    hr = x[:, DH:]
    for W, bias, relu in ((W1, b1, True), (W2, b2, True), (W3, b3, False)):
        pa, pb = _sc_aggregate(hl, hr, src_p, dst_p)
        xe, xo, ol, orr = _tc_linear(
            xe, xo, pa.reshape(N_ACC // 2, D), pb.reshape(N_ACC // 2, D),
            W, bias.reshape(1, D), relu)
        hl = ol.reshape(N_NODES, DH)
        hr = orr.reshape(N_NODES, DH)
    return jnp.concatenate(
        [xe[:, None, :], xo[:, None, :]], axis=1).reshape(N_NODES, D)
